# Initial kernel scaffold; baseline (speedup 1.0000x reference)
import jax
import jax.numpy as jnp
from jax.experimental import pallas as pl


def kernel(X, pixel_energy):
    raise NotImplementedError("write your pallas kernel here")


# speedup vs baseline: 1.7542x; 1.7542x over previous
import functools

import jax
import jax.numpy as jnp
from jax import lax
from jax.experimental import pallas as pl
from jax.experimental.pallas import tpu as pltpu
from jax.experimental.pallas import tpu_sc as plsc

H = 4096
W = 4096
N = 4194304

_NC = 2
_NS = 16
_NW = _NC * _NS
_NPW = N // _NW
_C = 2048
_NIT = _NPW // _C
_VR = _C // 16


def _sc_body(xx_hbm, xy_hbm, e_hbm, out_hbm, xs_v, ys_v, idx_v, g_v, o_v, sem):
    wid = lax.axis_index("s") * _NC + lax.axis_index("c")
    wbase = wid * _NPW

    def chunk(i, carry):
        base = wbase + i * _C
        pltpu.sync_copy(xx_hbm.at[pl.ds(base, _C)], xs_v)
        pltpu.sync_copy(xy_hbm.at[pl.ds(base, _C)], ys_v)

        def build(j, c):
            lane = j * 16
            x = xs_v[pl.ds(lane, 16)]
            y = ys_v[pl.ds(lane, 16)]
            sx = x * 2048.0 + 2048.0
            sy = y * 2048.0 + 2048.0
            ix = sx.astype(jnp.int32)
            iy = sy.astype(jnp.int32)
            ixc = jnp.clip(ix, 1, W - 2)
            iyc = jnp.clip(iy, 1, H - 2)
            flat = iyc * W + ixc
            idx_v[pl.ds(lane, 16)] = flat
            idx_v[pl.ds(_C + lane, 16)] = flat + 1
            idx_v[pl.ds(2 * _C + lane, 16)] = flat - 1
            idx_v[pl.ds(3 * _C + lane, 16)] = flat + W
            idx_v[pl.ds(4 * _C + lane, 16)] = flat - W
            return c

        lax.fori_loop(0, _VR, build, None)

        pltpu.async_copy(e_hbm.at[idx_v], g_v, sem).wait()

        def combine(j, c):
            lane = j * 16
            x = xs_v[pl.ds(lane, 16)]
            y = ys_v[pl.ds(lane, 16)]
            sx = x * 2048.0 + 2048.0
            sy = y * 2048.0 + 2048.0
            ix = sx.astype(jnp.int32)
            iy = sy.astype(jnp.int32)
            fx = sx - ix.astype(jnp.float32)
            fy = sy - iy.astype(jnp.float32)
            e0 = g_v[pl.ds(lane, 16)]
            exp_ = g_v[pl.ds(_C + lane, 16)]
            exm = g_v[pl.ds(2 * _C + lane, 16)]
            eyp = g_v[pl.ds(3 * _C + lane, 16)]
            eym = g_v[pl.ds(4 * _C + lane, 16)]
            dedx = 0.5 * (exp_ - exm)
            dedy = 0.5 * (eyp - eym)
            zero = jnp.float32(0.0)
            dx = jnp.maximum(jnp.maximum(-sx, zero),
                             jnp.maximum(sx - (W - 1), zero)) * (1.0 / 2048.0)
            dy = jnp.maximum(jnp.maximum(-sy, zero),
                             jnp.maximum(sy - (H - 1), zero)) * (1.0 / 2048.0)
            pen = dx * dx + dy * dy
            grad = fx * dedx + fy * dedy
            o_v[pl.ds(lane, 16)] = e0 + jnp.where(pen < 1e-6, grad, zero) + pen
            return c

        lax.fori_loop(0, _VR, combine, None)
        pltpu.sync_copy(o_v, out_hbm.at[pl.ds(base, _C)])
        return carry

    lax.fori_loop(0, _NIT, chunk, None)


_sc_image_energy = functools.partial(
    pl.kernel,
    mesh=plsc.VectorSubcoreMesh(core_axis_name="c", subcore_axis_name="s"),
    out_type=jax.ShapeDtypeStruct((N,), jnp.float32),
    scratch_types=[
        pltpu.VMEM((_C,), jnp.float32),
        pltpu.VMEM((_C,), jnp.float32),
        pltpu.VMEM((5 * _C,), jnp.int32),
        pltpu.VMEM((5 * _C,), jnp.float32),
        pltpu.VMEM((_C,), jnp.float32),
        pltpu.SemaphoreType.DMA,
    ],
)(_sc_body)


def kernel(X, pixel_energy):
    xx = X[:, 0]
    xy = X[:, 1]
    e = pixel_energy.reshape(-1)
    out = _sc_image_energy(xx, xy, e)
    return out[:, None]


# speedup vs baseline: 1.7632x; 1.0051x over previous
import functools

import jax
import jax.numpy as jnp
from jax import lax
from jax.experimental import pallas as pl
from jax.experimental.pallas import tpu as pltpu
from jax.experimental.pallas import tpu_sc as plsc

H = 4096
W = 4096
N = 4194304

_NC = 2
_NS = 16
_NW = _NC * _NS
_NPW = N // _NW
_C = 2048
_NIT = _NPW // _C
_VR = _C // 16


def _sc_body(xx_hbm, xy_hbm, e_hbm, out_hbm,
             xs_v, ys_v, idx0_v, idx1_v, g0_v, g1_v, o_v, sem0, sem1):
    wid = lax.axis_index("s") * _NC + lax.axis_index("c")
    wbase = wid * _NPW
    sems = (sem0, sem1)
    idxs = (idx0_v, idx1_v)
    gs = (g0_v, g1_v)

    def fire(i, slot):
        base = wbase + i * _C
        xsb, ysb, idxb, gb = xs_v.at[slot], ys_v.at[slot], idxs[slot], gs[slot]
        pltpu.sync_copy(xx_hbm.at[pl.ds(base, _C)], xsb)
        pltpu.sync_copy(xy_hbm.at[pl.ds(base, _C)], ysb)

        def build(j, c):
            lane = j * 16
            sx = xsb[pl.ds(lane, 16)] * 2048.0 + 2048.0
            sy = ysb[pl.ds(lane, 16)] * 2048.0 + 2048.0
            ixc = jnp.clip(sx.astype(jnp.int32), 1, W - 2)
            iyc = jnp.clip(sy.astype(jnp.int32), 1, H - 2)
            flat = iyc * W + ixc
            idxb[pl.ds(lane, 16)] = flat
            idxb[pl.ds(_C + lane, 16)] = flat + 1
            idxb[pl.ds(2 * _C + lane, 16)] = flat - 1
            idxb[pl.ds(3 * _C + lane, 16)] = flat + W
            idxb[pl.ds(4 * _C + lane, 16)] = flat - W
            return c

        lax.fori_loop(0, _VR, build, None)
        pltpu.async_copy(e_hbm.at[idxb], gb, sems[slot])

    def drain(i, slot):
        base = wbase + i * _C
        xsb, ysb, idxb, gb = xs_v.at[slot], ys_v.at[slot], idxs[slot], gs[slot]
        pltpu.make_async_copy(e_hbm.at[idxb], gb, sems[slot]).wait()

        def combine(j, c):
            lane = j * 16
            sx = xsb[pl.ds(lane, 16)] * 2048.0 + 2048.0
            sy = ysb[pl.ds(lane, 16)] * 2048.0 + 2048.0
            ix = sx.astype(jnp.int32)
            iy = sy.astype(jnp.int32)
            fx = sx - ix.astype(jnp.float32)
            fy = sy - iy.astype(jnp.float32)
            e0 = gb[pl.ds(lane, 16)]
            exp_ = gb[pl.ds(_C + lane, 16)]
            exm = gb[pl.ds(2 * _C + lane, 16)]
            eyp = gb[pl.ds(3 * _C + lane, 16)]
            eym = gb[pl.ds(4 * _C + lane, 16)]
            dedx = 0.5 * (exp_ - exm)
            dedy = 0.5 * (eyp - eym)
            zero = jnp.float32(0.0)
            dx = jnp.maximum(jnp.maximum(-sx, zero),
                             jnp.maximum(sx - (W - 1), zero)) * (1.0 / 2048.0)
            dy = jnp.maximum(jnp.maximum(-sy, zero),
                             jnp.maximum(sy - (H - 1), zero)) * (1.0 / 2048.0)
            pen = dx * dx + dy * dy
            grad = fx * dedx + fy * dedy
            o_v[pl.ds(lane, 16)] = e0 + jnp.where(pen < 1e-6, grad, zero) + pen
            return c

        lax.fori_loop(0, _VR, combine, None)
        pltpu.sync_copy(o_v, out_hbm.at[pl.ds(base, _C)])

    fire(0, 0)

    def outer(k, carry):
        i = 2 * k
        fire(i + 1, 1)
        drain(i, 0)
        fire(i + 2, 0)
        drain(i + 1, 1)
        return carry

    lax.fori_loop(0, _NIT // 2 - 1, outer, None)
    fire(_NIT - 1, 1)
    drain(_NIT - 2, 0)
    drain(_NIT - 1, 1)


_sc_image_energy = functools.partial(
    pl.kernel,
    mesh=plsc.VectorSubcoreMesh(core_axis_name="c", subcore_axis_name="s"),
    out_type=jax.ShapeDtypeStruct((N,), jnp.float32),
    scratch_types=[
        pltpu.VMEM((2, _C), jnp.float32),
        pltpu.VMEM((2, _C), jnp.float32),
        pltpu.VMEM((5 * _C,), jnp.int32),
        pltpu.VMEM((5 * _C,), jnp.int32),
        pltpu.VMEM((5 * _C,), jnp.float32),
        pltpu.VMEM((5 * _C,), jnp.float32),
        pltpu.VMEM((_C,), jnp.float32),
        pltpu.SemaphoreType.DMA,
        pltpu.SemaphoreType.DMA,
    ],
)(_sc_body)


def kernel(X, pixel_energy):
    xx = X[:, 0]
    xy = X[:, 1]
    e = pixel_energy.reshape(-1)
    out = _sc_image_energy(xx, xy, e)
    return out[:, None]


# speedup vs baseline: 1.7640x; 1.0004x over previous
import functools

import jax
import jax.numpy as jnp
from jax import lax
from jax.experimental import pallas as pl
from jax.experimental.pallas import tpu as pltpu
from jax.experimental.pallas import tpu_sc as plsc

H = 4096
W = 4096
N = 4194304

_NC = 2
_NS = 16
_NW = _NC * _NS
_NPW = N // _NW
_C = 2048
_NIT = _NPW // _C
_VR = _C // 16


def _sc_body(xx_hbm, xy_hbm, e_hbm, out_hbm,
             xs_v, ys_v,
             ia0, ib0, ic0, id0, ie0,
             ia1, ib1, ic1, id1, ie1,
             ga0, gb0, gc0, gd0, ge0,
             ga1, gb1, gc1, gd1, ge1,
             o_v, sem0, sem1):
    wid = lax.axis_index("s") * _NC + lax.axis_index("c")
    wbase = wid * _NPW
    sems = (sem0, sem1)
    idxs = ((ia0, ib0, ic0, id0, ie0), (ia1, ib1, ic1, id1, ie1))
    gs = ((ga0, gb0, gc0, gd0, ge0), (ga1, gb1, gc1, gd1, ge1))

    def fire(i, slot):
        base = wbase + i * _C
        xsb, ysb = xs_v.at[slot], ys_v.at[slot]
        ib = idxs[slot]
        pltpu.sync_copy(xx_hbm.at[pl.ds(base, _C)], xsb)
        pltpu.sync_copy(xy_hbm.at[pl.ds(base, _C)], ysb)

        def build(j, c):
            lane = j * 16
            sx = xsb[pl.ds(lane, 16)] * 2048.0 + 2048.0
            sy = ysb[pl.ds(lane, 16)] * 2048.0 + 2048.0
            ixc = jnp.clip(sx.astype(jnp.int32), 1, W - 2)
            iyc = jnp.clip(sy.astype(jnp.int32), 1, H - 2)
            flat = iyc * W + ixc
            ib[0][pl.ds(lane, 16)] = flat
            ib[1][pl.ds(lane, 16)] = flat + 1
            ib[2][pl.ds(lane, 16)] = flat - 1
            ib[3][pl.ds(lane, 16)] = flat + W
            ib[4][pl.ds(lane, 16)] = flat - W
            return c

        lax.fori_loop(0, _VR, build, None)
        for k in range(5):
            pltpu.async_copy(e_hbm.at[ib[k]], gs[slot][k], sems[slot])

    def drain(i, slot):
        base = wbase + i * _C
        xsb, ysb = xs_v.at[slot], ys_v.at[slot]
        for k in range(5):
            pltpu.make_async_copy(e_hbm.at[idxs[slot][k]],
                                  gs[slot][k], sems[slot]).wait()

        def combine(j, c):
            lane = j * 16
            sx = xsb[pl.ds(lane, 16)] * 2048.0 + 2048.0
            sy = ysb[pl.ds(lane, 16)] * 2048.0 + 2048.0
            ix = sx.astype(jnp.int32)
            iy = sy.astype(jnp.int32)
            fx = sx - ix.astype(jnp.float32)
            fy = sy - iy.astype(jnp.float32)
            e0 = gs[slot][0][pl.ds(lane, 16)]
            exp_ = gs[slot][1][pl.ds(lane, 16)]
            exm = gs[slot][2][pl.ds(lane, 16)]
            eyp = gs[slot][3][pl.ds(lane, 16)]
            eym = gs[slot][4][pl.ds(lane, 16)]
            dedx = 0.5 * (exp_ - exm)
            dedy = 0.5 * (eyp - eym)
            zero = jnp.float32(0.0)
            dx = jnp.maximum(jnp.maximum(-sx, zero),
                             jnp.maximum(sx - (W - 1), zero)) * (1.0 / 2048.0)
            dy = jnp.maximum(jnp.maximum(-sy, zero),
                             jnp.maximum(sy - (H - 1), zero)) * (1.0 / 2048.0)
            pen = dx * dx + dy * dy
            grad = fx * dedx + fy * dedy
            o_v[pl.ds(lane, 16)] = e0 + jnp.where(pen < 1e-6, grad, zero) + pen
            return c

        lax.fori_loop(0, _VR, combine, None)
        pltpu.sync_copy(o_v, out_hbm.at[pl.ds(base, _C)])

    fire(0, 0)

    def outer(k, carry):
        i = 2 * k
        fire(i + 1, 1)
        drain(i, 0)
        fire(i + 2, 0)
        drain(i + 1, 1)
        return carry

    lax.fori_loop(0, _NIT // 2 - 1, outer, None)
    fire(_NIT - 1, 1)
    drain(_NIT - 2, 0)
    drain(_NIT - 1, 1)


_sc_image_energy = functools.partial(
    pl.kernel,
    mesh=plsc.VectorSubcoreMesh(core_axis_name="c", subcore_axis_name="s"),
    out_type=jax.ShapeDtypeStruct((N,), jnp.float32),
    scratch_types=[
        pltpu.VMEM((2, _C), jnp.float32),
        pltpu.VMEM((2, _C), jnp.float32),
    ] + [pltpu.VMEM((_C,), jnp.int32) for _ in range(10)
    ] + [pltpu.VMEM((_C,), jnp.float32) for _ in range(10)
    ] + [
        pltpu.VMEM((_C,), jnp.float32),
        pltpu.SemaphoreType.DMA,
        pltpu.SemaphoreType.DMA,
    ],
)(_sc_body)


def kernel(X, pixel_energy):
    xx = X[:, 0]
    xy = X[:, 1]
    e = pixel_energy.reshape(-1)
    out = _sc_image_energy(xx, xy, e)
    return out[:, None]
